# baseline (speedup 1.0000x reference)
import jax
import jax.numpy as jnp
from jax.experimental import pallas as pl
from jax.experimental.pallas import tpu as pltpu

D_MODEL_ = 2048
N_EXP_ = 64
BM_ = 512


def _router_body(x_ref, w_ref, b_ref, o_ref):
    acc = jax.lax.dot_general(
        w_ref[...],
        x_ref[0],
        (((1,), (1,)), ((), ())),
        preferred_element_type=jnp.float32,
    )
    o_ref[0] = acc + b_ref[...].reshape(N_EXP_, 1)


def kernel(x, W, b):
    bsz, seq, d = x.shape
    grid = (bsz, seq // BM_)
    out = pl.pallas_call(
        _router_body,
        grid=grid,
        in_specs=[
            pl.BlockSpec((1, BM_, d), lambda i, j: (i, j, 0)),
            pl.BlockSpec((N_EXP_, d), lambda i, j: (0, 0)),
            pl.BlockSpec((N_EXP_,), lambda i, j: (0,)),
        ],
        out_specs=pl.BlockSpec((1, N_EXP_, BM_), lambda i, j: (i, 0, j)),
        out_shape=jax.ShapeDtypeStruct((bsz, N_EXP_, seq), jnp.float32),
        compiler_params=pltpu.CompilerParams(
            dimension_semantics=("arbitrary", "arbitrary"),
        ),
    )(x, W, b)
    return jnp.swapaxes(out, 1, 2)


# speedup vs baseline: 1.1177x; 1.1177x over previous
import functools

import jax
import jax.numpy as jnp
from jax.experimental import pallas as pl
from jax.experimental.pallas import tpu as pltpu

D_MODEL_ = 2048
N_EXP_ = 64
BM_ = 1024
NBUF_ = 4


def _router_body(x_hbm, w_ref, b_ref, o_ref, xbuf, sem, *, n_steps, seq_blocks):
    def copy(i):
        return pltpu.make_async_copy(
            x_hbm.at[pl.ds(i // seq_blocks, 1), pl.ds((i % seq_blocks) * BM_, BM_), :],
            xbuf.at[i % NBUF_],
            sem.at[i % NBUF_],
        )

    for i in range(min(NBUF_, n_steps)):
        copy(i).start()
    w = w_ref[...]
    bias = b_ref[...].reshape(N_EXP_, 1)
    dn = (((1,), (1,)), ((), ()))
    for i in range(n_steps):
        copy(i).wait()
        acc = jax.lax.dot_general(
            w, xbuf[i % NBUF_, 0], dn, preferred_element_type=jnp.float32
        )
        o_ref[i // seq_blocks, :, pl.ds((i % seq_blocks) * BM_, BM_)] = acc + bias
        if i + NBUF_ < n_steps:
            copy(i + NBUF_).start()


def kernel(x, W, b):
    bsz, seq, d = x.shape
    seq_blocks = seq // BM_
    n_steps = bsz * seq_blocks
    out = pl.pallas_call(
        functools.partial(_router_body, n_steps=n_steps, seq_blocks=seq_blocks),
        in_specs=[
            pl.BlockSpec(memory_space=pltpu.MemorySpace.HBM),
            pl.BlockSpec(memory_space=pltpu.VMEM),
            pl.BlockSpec(memory_space=pltpu.VMEM),
        ],
        out_specs=pl.BlockSpec(memory_space=pltpu.VMEM),
        out_shape=jax.ShapeDtypeStruct((bsz, N_EXP_, seq), jnp.float32),
        scratch_shapes=[
            pltpu.VMEM((NBUF_, 1, BM_, d), jnp.float32),
            pltpu.SemaphoreType.DMA((NBUF_,)),
        ],
    )(x, W, b)
    return jnp.swapaxes(out, 1, 2)


# speedup vs baseline: 1.1493x; 1.0282x over previous
import jax
import jax.numpy as jnp
from jax.experimental import pallas as pl
from jax.experimental.pallas import tpu as pltpu

D_MODEL_ = 2048
N_EXP_ = 64
BM_ = 1024


def _router_body(x_ref, w_ref, b_ref, o_ref):
    acc = jax.lax.dot_general(
        w_ref[...],
        x_ref[0],
        (((1,), (1,)), ((), ())),
        preferred_element_type=jnp.float32,
    )
    o_ref[0] = acc + b_ref[...]


def kernel(x, W, b):
    bsz, seq, d = x.shape
    grid = (bsz, seq // BM_)
    out = pl.pallas_call(
        _router_body,
        grid=grid,
        in_specs=[
            pl.BlockSpec((1, BM_, d), lambda i, j: (i, j, 0)),
            pl.BlockSpec((N_EXP_, d), lambda i, j: (0, 0)),
            pl.BlockSpec((N_EXP_, 1), lambda i, j: (0, 0)),
        ],
        out_specs=pl.BlockSpec((1, N_EXP_, BM_), lambda i, j: (i, 0, j)),
        out_shape=jax.ShapeDtypeStruct((bsz, N_EXP_, seq), jnp.float32),
        compiler_params=pltpu.CompilerParams(
            dimension_semantics=("arbitrary", "arbitrary"),
        ),
    )(x, W, b.reshape(N_EXP_, 1))
    return jnp.swapaxes(out, 1, 2)


# speedup vs baseline: 1.1881x; 1.0337x over previous
import jax
import jax.numpy as jnp
from jax.experimental import pallas as pl
from jax.experimental.pallas import tpu as pltpu

D_MODEL_ = 2048
N_EXP_ = 64
BM_ = 1024


def _router_body(x_ref, w_ref, b_ref, o_ref):
    acc = jax.lax.dot_general(
        w_ref[...],
        x_ref[0],
        (((1,), (1,)), ((), ())),
        preferred_element_type=jnp.float32,
    )
    o_ref[0] = acc + b_ref[...].reshape(N_EXP_, 1)


def kernel(x, W, b):
    bsz, seq, d = x.shape
    grid = (bsz, seq // BM_)
    out = pl.pallas_call(
        _router_body,
        grid=grid,
        in_specs=[
            pl.BlockSpec((1, BM_, d), lambda i, j: (i, j, 0)),
            pl.BlockSpec((N_EXP_, d), lambda i, j: (0, 0)),
            pl.BlockSpec((N_EXP_,), lambda i, j: (0,)),
        ],
        out_specs=pl.BlockSpec((1, N_EXP_, BM_), lambda i, j: (i, 0, j)),
        out_shape=jax.ShapeDtypeStruct((bsz, N_EXP_, seq), jnp.float32),
        compiler_params=pltpu.CompilerParams(
            dimension_semantics=("arbitrary", "arbitrary"),
            fuse_transposed_lhs_in_matmul=True,
        ),
    )(x, W, b)
    return jnp.swapaxes(out, 1, 2)
